# baseline (device time: 340555 ns/iter reference)
import jax
import jax.numpy as jnp
from jax import lax
from jax.experimental import pallas as pl
from jax.experimental.pallas import tpu as pltpu

N_DEV = 16
N_PEERS = N_DEV - 1


def kernel(x, w_mat):
    m_per, k = x.shape
    n_per = w_mat.shape[1]

    def body(x_ref, w_ref, out_ref,
             colrecv, bufA_fp, bufA_fn, bufB, amax_cur, amax_all,
             sendUp, sendDn, recvC, sendAc, recvA_fp, sendAcc, recvA_fn,
             sendB, recvB, bsend, brecv, creditB0, creditB1):
        me = lax.axis_index("i")
        q = lax.rem(me, 4)
        z = me // 4
        nxt = 4 * z + lax.rem(q + 1, 4)
        prv = 4 * z + lax.rem(q + 3, 4)

        zz = [
            z,
            jnp.where(z == 0, 1, jnp.where(z == 1, 0, jnp.where(z == 2, 1, 2))),
            jnp.where(z == 0, 2, jnp.where(z == 1, 2, jnp.where(z == 2, 3, 1))),
            jnp.where(z == 0, 3, jnp.where(z == 1, 3, jnp.where(z == 2, 0, 0))),
        ]

        def col_slot(origin_z):
            return jnp.where(origin_z < z, origin_z, origin_z - 1)

        def gemm_into(origin_pos, xc, local_max):
            y = lax.dot_general(
                xc, w_ref[...], (((1,), (0,)), ((), ())),
                preferred_element_type=jnp.float32,
                precision=lax.Precision.HIGHEST,
            )
            y = jnp.maximum(y, 0.0)
            out_ref[pl.ds(origin_pos * m_per, m_per), :] = y
            return jnp.maximum(local_max, jnp.max(y))

        def start_stepA(w, src):
            rd_cw = pltpu.make_async_remote_copy(
                src_ref=src, dst_ref=bufA_fp.at[w],
                send_sem=sendAc.at[w], recv_sem=recvA_fp.at[w],
                device_id=(nxt,), device_id_type=pl.DeviceIdType.MESH,
            )
            rd_ccw = pltpu.make_async_remote_copy(
                src_ref=src, dst_ref=bufA_fn.at[w],
                send_sem=sendAcc.at[w], recv_sem=recvA_fn.at[w],
                device_id=(prv,), device_id_type=pl.DeviceIdType.MESH,
            )
            rd_cw.start()
            rd_ccw.start()
            return rd_cw, rd_ccw

        def start_stepB(w):
            if w % 2 == 0:
                src, tgt = bufA_fp.at[w], nxt
            else:
                src, tgt = bufA_fn.at[w], prv
            rd = pltpu.make_async_remote_copy(
                src_ref=src, dst_ref=bufB.at[w % 2],
                send_sem=sendB.at[w], recv_sem=recvB.at[w],
                device_id=(tgt,), device_id_type=pl.DeviceIdType.MESH,
            )
            rd.start()
            return rd

        def col_wait(j):
            slot = col_slot(zz[j])
            rd = pltpu.make_async_remote_copy(
                src_ref=x_ref, dst_ref=colrecv.at[slot],
                send_sem=sendUp.at[0], recv_sem=recvC.at[slot],
                device_id=(nxt,), device_id_type=pl.DeviceIdType.MESH,
            )
            rd.wait_recv()

        up_tgt = (4 * jnp.minimum(z + 1, 3) + q,)
        dn_tgt = (4 * jnp.maximum(z - 1, 0) + q,)

        def mk_up(j, src, slot):
            return pltpu.make_async_remote_copy(
                src_ref=src, dst_ref=colrecv.at[slot],
                send_sem=sendUp.at[j], recv_sem=recvC.at[slot],
                device_id=up_tgt, device_id_type=pl.DeviceIdType.MESH,
            )

        def mk_dn(j, src, slot):
            return pltpu.make_async_remote_copy(
                src_ref=src, dst_ref=colrecv.at[slot],
                send_sem=sendDn.at[j], recv_sem=recvC.at[slot],
                device_id=dn_tgt, device_id_type=pl.DeviceIdType.MESH,
            )

        def mk_fwd(j):
            c = zz[j]
            up = mk_up(j, colrecv.at[col_slot(c)], jnp.minimum(c, 2))
            dn = mk_dn(j, colrecv.at[col_slot(c)], jnp.maximum(c - 1, 0))
            cu = jnp.logical_and(c < z, z < 3)
            cd = jnp.logical_and(c > z, z > 0)
            return up, dn, cu, cd

        def fwd_start(fwd):
            up, dn, cu, cd = fwd

            @pl.when(cu)
            def _():
                up.start()

            @pl.when(cd)
            def _():
                dn.start()

        def fwd_drain(fwd):
            up, dn, cu, cd = fwd

            @pl.when(cu)
            def _():
                up.wait_send()

            @pl.when(cd)
            def _():
                dn.wait_send()

        def sig(sem, tgt):
            pl.semaphore_signal(sem, inc=1, device_id=(tgt,),
                                device_id_type=pl.DeviceIdType.MESH)

        barrier_sem = pltpu.get_barrier_semaphore()
        sig(barrier_sem, nxt)
        sig(barrier_sem, prv)

        @pl.when(z < 3)
        def _():
            pl.semaphore_signal(barrier_sem, inc=1, device_id=up_tgt,
                                device_id_type=pl.DeviceIdType.MESH)

        @pl.when(z > 0)
        def _():
            pl.semaphore_signal(barrier_sem, inc=1, device_id=dn_tgt,
                                device_id_type=pl.DeviceIdType.MESH)

        nsigs = (2 + jnp.where(z > 0, 1, 0) + jnp.where(z < 3, 1, 0))
        pl.semaphore_wait(barrier_sem, nsigs)

        up_own = mk_up(0, x_ref, jnp.minimum(z, 2))
        dn_own = mk_dn(0, x_ref, jnp.maximum(z - 1, 0))

        @pl.when(z < 3)
        def _():
            up_own.start()

        @pl.when(z > 0)
        def _():
            dn_own.start()

        a0_cw, a0_ccw = start_stepA(0, x_ref)
        local_max = gemm_into(me, x_ref[...], jnp.float32(0.0))

        col_wait(1)
        fwd1 = mk_fwd(1)
        fwd_start(fwd1)
        interior = jnp.logical_and(z >= 1, z <= 2)
        fwd2 = mk_fwd(2)

        @pl.when(interior)
        def _():
            col_wait(2)
            fwd_start(fwd2)

        a0_cw.wait_recv()
        a0_ccw.wait_recv()
        a1_cw, a1_ccw = start_stepA(1, colrecv.at[col_slot(zz[1])])
        local_max = gemm_into(4 * z + lax.rem(q + 3, 4),
                              bufA_fp[0, :, :], local_max)
        local_max = gemm_into(4 * z + lax.rem(q + 1, 4),
                              bufA_fn[0, :, :], local_max)
        local_max = gemm_into(4 * zz[1] + q,
                              colrecv[col_slot(zz[1]), :, :], local_max)

        @pl.when(jnp.logical_not(interior))
        def _():
            col_wait(2)

        a1_cw.wait_recv()
        a1_ccw.wait_recv()
        a2_cw, a2_ccw = start_stepA(2, colrecv.at[col_slot(zz[2])])
        local_max = gemm_into(4 * zz[1] + lax.rem(q + 3, 4),
                              bufA_fp[1, :, :], local_max)
        local_max = gemm_into(4 * zz[1] + lax.rem(q + 1, 4),
                              bufA_fn[1, :, :], local_max)
        local_max = gemm_into(4 * zz[2] + q,
                              colrecv[col_slot(zz[2]), :, :], local_max)

        col_wait(3)
        fwd3 = mk_fwd(3)
        fwd_start(fwd3)
        a2_cw.wait_recv()
        a2_ccw.wait_recv()
        a3_cw, a3_ccw = start_stepA(3, colrecv.at[col_slot(zz[3])])
        local_max = gemm_into(4 * zz[2] + lax.rem(q + 3, 4),
                              bufA_fp[2, :, :], local_max)
        local_max = gemm_into(4 * zz[2] + lax.rem(q + 1, 4),
                              bufA_fn[2, :, :], local_max)
        local_max = gemm_into(4 * zz[3] + q,
                              colrecv[col_slot(zz[3]), :, :], local_max)

        a3_cw.wait_recv()
        a3_ccw.wait_recv()
        local_max = gemm_into(4 * zz[3] + lax.rem(q + 3, 4),
                              bufA_fp[3, :, :], local_max)
        local_max = gemm_into(4 * zz[3] + lax.rem(q + 1, 4),
                              bufA_fn[3, :, :], local_max)

        b0 = start_stepB(0)
        b1 = start_stepB(1)
        b0.wait_recv()
        local_max = gemm_into(4 * z + lax.rem(q + 2, 4),
                              bufB[0, :, :], local_max)
        sig(creditB0, prv)
        pl.semaphore_wait(creditB0, 1)
        b2 = start_stepB(2)
        b1.wait_recv()
        local_max = gemm_into(4 * zz[1] + lax.rem(q + 2, 4),
                              bufB[1, :, :], local_max)
        sig(creditB1, nxt)
        pl.semaphore_wait(creditB1, 1)
        b3 = start_stepB(3)
        b2.wait_recv()
        local_max = gemm_into(4 * zz[2] + lax.rem(q + 2, 4),
                              bufB[0, :, :], local_max)
        b3.wait_recv()
        local_max = gemm_into(4 * zz[3] + lax.rem(q + 2, 4),
                              bufB[1, :, :], local_max)

        @pl.when(z < 3)
        def _():
            up_own.wait_send()

        @pl.when(z > 0)
        def _():
            dn_own.wait_send()

        for fwd in (fwd1, fwd2, fwd3):
            fwd_drain(fwd)
        for rd in (a0_cw, a0_ccw, a1_cw, a1_ccw, a2_cw, a2_ccw,
                   a3_cw, a3_ccw, b0, b1, b2, b3):
            rd.wait_send()

        amax_cur[...] = jnp.full((8, 128), local_max, jnp.float32)
        brd = []
        for d in range(1, N_DEV):
            tgt = lax.rem(me + d, N_DEV)
            rd = pltpu.make_async_remote_copy(
                src_ref=amax_cur, dst_ref=amax_all.at[d - 1],
                send_sem=bsend.at[d - 1], recv_sem=brecv.at[d - 1],
                device_id=(tgt,), device_id_type=pl.DeviceIdType.MESH,
            )
            rd.start()
            brd.append(rd)
        gmax = local_max
        for d in range(1, N_DEV):
            brd[d - 1].wait_recv()
            gmax = jnp.maximum(gmax, amax_all[d - 1, 0, 0])
        for d in range(1, N_DEV):
            brd[d - 1].wait_send()

        scale = gmax / 448.0
        qv = (out_ref[...] / scale).astype(jnp.float8_e4m3fn)
        out_ref[...] = qv.astype(jnp.float32) * scale

    return pl.pallas_call(
        body,
        out_shape=jax.ShapeDtypeStruct((N_DEV * m_per, n_per), jnp.float32),
        in_specs=[
            pl.BlockSpec(memory_space=pltpu.VMEM),
            pl.BlockSpec(memory_space=pltpu.VMEM),
        ],
        out_specs=pl.BlockSpec(memory_space=pltpu.VMEM),
        scratch_shapes=[
            pltpu.VMEM((3, m_per, k), jnp.float32),
            pltpu.VMEM((4, m_per, k), jnp.float32),
            pltpu.VMEM((4, m_per, k), jnp.float32),
            pltpu.VMEM((2, m_per, k), jnp.float32),
            pltpu.VMEM((8, 128), jnp.float32),
            pltpu.VMEM((N_PEERS, 8, 128), jnp.float32),
            pltpu.SemaphoreType.DMA((4,)),
            pltpu.SemaphoreType.DMA((4,)),
            pltpu.SemaphoreType.DMA((3,)),
            pltpu.SemaphoreType.DMA((4,)),
            pltpu.SemaphoreType.DMA((4,)),
            pltpu.SemaphoreType.DMA((4,)),
            pltpu.SemaphoreType.DMA((4,)),
            pltpu.SemaphoreType.DMA((4,)),
            pltpu.SemaphoreType.DMA((4,)),
            pltpu.SemaphoreType.DMA((N_PEERS,)),
            pltpu.SemaphoreType.DMA((N_PEERS,)),
            pltpu.SemaphoreType.REGULAR,
            pltpu.SemaphoreType.REGULAR,
        ],
        compiler_params=pltpu.CompilerParams(
            vmem_limit_bytes=63 * 1024 * 1024,
            collective_id=0,
        ),
    )(x, w_mat)


# device time: 169427 ns/iter; 2.0100x vs baseline; 2.0100x over previous
import jax
import jax.numpy as jnp
from jax import lax
from jax.experimental import pallas as pl
from jax.experimental.pallas import tpu as pltpu

N_DEV = 16
N_PEERS = N_DEV - 1


def kernel(x, w_mat):
    m_per, k = x.shape
    n_per = w_mat.shape[1]

    def body(x_ref, w_ref, out_ref,
             xb, wb, colrecv, bufA_fp, bufA_fn, bufB, amax_cur, amax_all,
             sendUp, sendDn, recvC, sendAc, recvA_fp, sendAcc, recvA_fn,
             sendB, recvB, bsend, brecv, creditB0, creditB1):
        me = lax.axis_index("i")
        q = lax.rem(me, 4)
        z = me // 4
        nxt = 4 * z + lax.rem(q + 1, 4)
        prv = 4 * z + lax.rem(q + 3, 4)

        zz = [
            z,
            jnp.where(z == 0, 1, jnp.where(z == 1, 0, jnp.where(z == 2, 1, 2))),
            jnp.where(z == 0, 2, jnp.where(z == 1, 2, jnp.where(z == 2, 3, 1))),
            jnp.where(z == 0, 3, jnp.where(z == 1, 3, jnp.where(z == 2, 0, 0))),
        ]

        def col_slot(origin_z):
            return jnp.where(origin_z < z, origin_z, origin_z - 1)

        def gemm_into(origin_pos, xc, local_max):
            y = lax.dot_general(
                xc, wb[...], (((1,), (0,)), ((), ())),
                preferred_element_type=jnp.float32,
            )
            y = jnp.maximum(y, 0.0)
            out_ref[pl.ds(origin_pos * m_per, m_per), :] = y
            return jnp.maximum(local_max, jnp.max(y))

        def start_stepA(w, src):
            rd_cw = pltpu.make_async_remote_copy(
                src_ref=src, dst_ref=bufA_fp.at[w],
                send_sem=sendAc.at[w], recv_sem=recvA_fp.at[w],
                device_id=(nxt,), device_id_type=pl.DeviceIdType.MESH,
            )
            rd_ccw = pltpu.make_async_remote_copy(
                src_ref=src, dst_ref=bufA_fn.at[w],
                send_sem=sendAcc.at[w], recv_sem=recvA_fn.at[w],
                device_id=(prv,), device_id_type=pl.DeviceIdType.MESH,
            )
            rd_cw.start()
            rd_ccw.start()
            return rd_cw, rd_ccw

        def start_stepB(w):
            if w % 2 == 0:
                src, tgt = bufA_fp.at[w], nxt
            else:
                src, tgt = bufA_fn.at[w], prv
            rd = pltpu.make_async_remote_copy(
                src_ref=src, dst_ref=bufB.at[w % 2],
                send_sem=sendB.at[w], recv_sem=recvB.at[w],
                device_id=(tgt,), device_id_type=pl.DeviceIdType.MESH,
            )
            rd.start()
            return rd

        def col_wait(j):
            slot = col_slot(zz[j])
            rd = pltpu.make_async_remote_copy(
                src_ref=xb, dst_ref=colrecv.at[slot],
                send_sem=sendUp.at[0], recv_sem=recvC.at[slot],
                device_id=(nxt,), device_id_type=pl.DeviceIdType.MESH,
            )
            rd.wait_recv()

        up_tgt = (4 * jnp.minimum(z + 1, 3) + q,)
        dn_tgt = (4 * jnp.maximum(z - 1, 0) + q,)

        def mk_up(j, src, slot):
            return pltpu.make_async_remote_copy(
                src_ref=src, dst_ref=colrecv.at[slot],
                send_sem=sendUp.at[j], recv_sem=recvC.at[slot],
                device_id=up_tgt, device_id_type=pl.DeviceIdType.MESH,
            )

        def mk_dn(j, src, slot):
            return pltpu.make_async_remote_copy(
                src_ref=src, dst_ref=colrecv.at[slot],
                send_sem=sendDn.at[j], recv_sem=recvC.at[slot],
                device_id=dn_tgt, device_id_type=pl.DeviceIdType.MESH,
            )

        def mk_fwd(j):
            c = zz[j]
            up = mk_up(j, colrecv.at[col_slot(c)], jnp.minimum(c, 2))
            dn = mk_dn(j, colrecv.at[col_slot(c)], jnp.maximum(c - 1, 0))
            cu = jnp.logical_and(c < z, z < 3)
            cd = jnp.logical_and(c > z, z > 0)
            return up, dn, cu, cd

        def fwd_start(fwd):
            up, dn, cu, cd = fwd

            @pl.when(cu)
            def _():
                up.start()

            @pl.when(cd)
            def _():
                dn.start()

        def fwd_drain(fwd):
            up, dn, cu, cd = fwd

            @pl.when(cu)
            def _():
                up.wait_send()

            @pl.when(cd)
            def _():
                dn.wait_send()

        def sig(sem, tgt):
            pl.semaphore_signal(sem, inc=1, device_id=(tgt,),
                                device_id_type=pl.DeviceIdType.MESH)

        xb[...] = x_ref[...].astype(jnp.bfloat16)
        wb[...] = w_ref[...].astype(jnp.bfloat16)

        barrier_sem = pltpu.get_barrier_semaphore()
        sig(barrier_sem, nxt)
        sig(barrier_sem, prv)

        @pl.when(z < 3)
        def _():
            pl.semaphore_signal(barrier_sem, inc=1, device_id=up_tgt,
                                device_id_type=pl.DeviceIdType.MESH)

        @pl.when(z > 0)
        def _():
            pl.semaphore_signal(barrier_sem, inc=1, device_id=dn_tgt,
                                device_id_type=pl.DeviceIdType.MESH)

        nsigs = (2 + jnp.where(z > 0, 1, 0) + jnp.where(z < 3, 1, 0))
        pl.semaphore_wait(barrier_sem, nsigs)

        up_own = mk_up(0, xb, jnp.minimum(z, 2))
        dn_own = mk_dn(0, xb, jnp.maximum(z - 1, 0))

        @pl.when(z < 3)
        def _():
            up_own.start()

        @pl.when(z > 0)
        def _():
            dn_own.start()

        a0_cw, a0_ccw = start_stepA(0, xb)
        local_max = gemm_into(me, xb[...], jnp.float32(0.0))

        col_wait(1)
        fwd1 = mk_fwd(1)
        fwd_start(fwd1)
        interior = jnp.logical_and(z >= 1, z <= 2)
        fwd2 = mk_fwd(2)

        @pl.when(interior)
        def _():
            col_wait(2)
            fwd_start(fwd2)

        a0_cw.wait_recv()
        a0_ccw.wait_recv()
        a1_cw, a1_ccw = start_stepA(1, colrecv.at[col_slot(zz[1])])
        local_max = gemm_into(4 * z + lax.rem(q + 3, 4),
                              bufA_fp[0, :, :], local_max)
        local_max = gemm_into(4 * z + lax.rem(q + 1, 4),
                              bufA_fn[0, :, :], local_max)
        local_max = gemm_into(4 * zz[1] + q,
                              colrecv[col_slot(zz[1]), :, :], local_max)

        @pl.when(jnp.logical_not(interior))
        def _():
            col_wait(2)

        a1_cw.wait_recv()
        a1_ccw.wait_recv()
        a2_cw, a2_ccw = start_stepA(2, colrecv.at[col_slot(zz[2])])
        local_max = gemm_into(4 * zz[1] + lax.rem(q + 3, 4),
                              bufA_fp[1, :, :], local_max)
        local_max = gemm_into(4 * zz[1] + lax.rem(q + 1, 4),
                              bufA_fn[1, :, :], local_max)
        local_max = gemm_into(4 * zz[2] + q,
                              colrecv[col_slot(zz[2]), :, :], local_max)

        col_wait(3)
        fwd3 = mk_fwd(3)
        fwd_start(fwd3)
        a2_cw.wait_recv()
        a2_ccw.wait_recv()
        a3_cw, a3_ccw = start_stepA(3, colrecv.at[col_slot(zz[3])])
        local_max = gemm_into(4 * zz[2] + lax.rem(q + 3, 4),
                              bufA_fp[2, :, :], local_max)
        local_max = gemm_into(4 * zz[2] + lax.rem(q + 1, 4),
                              bufA_fn[2, :, :], local_max)
        local_max = gemm_into(4 * zz[3] + q,
                              colrecv[col_slot(zz[3]), :, :], local_max)

        a3_cw.wait_recv()
        a3_ccw.wait_recv()
        local_max = gemm_into(4 * zz[3] + lax.rem(q + 3, 4),
                              bufA_fp[3, :, :], local_max)
        local_max = gemm_into(4 * zz[3] + lax.rem(q + 1, 4),
                              bufA_fn[3, :, :], local_max)

        b0 = start_stepB(0)
        b1 = start_stepB(1)
        b0.wait_recv()
        local_max = gemm_into(4 * z + lax.rem(q + 2, 4),
                              bufB[0, :, :], local_max)
        sig(creditB0, prv)
        pl.semaphore_wait(creditB0, 1)
        b2 = start_stepB(2)
        b1.wait_recv()
        local_max = gemm_into(4 * zz[1] + lax.rem(q + 2, 4),
                              bufB[1, :, :], local_max)
        sig(creditB1, nxt)
        pl.semaphore_wait(creditB1, 1)
        b3 = start_stepB(3)
        b2.wait_recv()
        local_max = gemm_into(4 * zz[2] + lax.rem(q + 2, 4),
                              bufB[0, :, :], local_max)
        b3.wait_recv()
        local_max = gemm_into(4 * zz[3] + lax.rem(q + 2, 4),
                              bufB[1, :, :], local_max)

        @pl.when(z < 3)
        def _():
            up_own.wait_send()

        @pl.when(z > 0)
        def _():
            dn_own.wait_send()

        for fwd in (fwd1, fwd2, fwd3):
            fwd_drain(fwd)
        for rd in (a0_cw, a0_ccw, a1_cw, a1_ccw, a2_cw, a2_ccw,
                   a3_cw, a3_ccw, b0, b1, b2, b3):
            rd.wait_send()

        amax_cur[...] = jnp.full((8, 128), local_max, jnp.float32)
        brd = []
        for d in range(1, N_DEV):
            tgt = lax.rem(me + d, N_DEV)
            rd = pltpu.make_async_remote_copy(
                src_ref=amax_cur, dst_ref=amax_all.at[d - 1],
                send_sem=bsend.at[d - 1], recv_sem=brecv.at[d - 1],
                device_id=(tgt,), device_id_type=pl.DeviceIdType.MESH,
            )
            rd.start()
            brd.append(rd)
        gmax = local_max
        for d in range(1, N_DEV):
            brd[d - 1].wait_recv()
            gmax = jnp.maximum(gmax, amax_all[d - 1, 0, 0])
        for d in range(1, N_DEV):
            brd[d - 1].wait_send()

        scale = gmax / 448.0
        qv = (out_ref[...] / scale).astype(jnp.float8_e4m3fn)
        out_ref[...] = qv.astype(jnp.float32) * scale

    return pl.pallas_call(
        body,
        out_shape=jax.ShapeDtypeStruct((N_DEV * m_per, n_per), jnp.float32),
        in_specs=[
            pl.BlockSpec(memory_space=pltpu.VMEM),
            pl.BlockSpec(memory_space=pltpu.VMEM),
        ],
        out_specs=pl.BlockSpec(memory_space=pltpu.VMEM),
        scratch_shapes=[
            pltpu.VMEM((m_per, k), jnp.bfloat16),
            pltpu.VMEM((k, n_per), jnp.bfloat16),
            pltpu.VMEM((3, m_per, k), jnp.bfloat16),
            pltpu.VMEM((4, m_per, k), jnp.bfloat16),
            pltpu.VMEM((4, m_per, k), jnp.bfloat16),
            pltpu.VMEM((2, m_per, k), jnp.bfloat16),
            pltpu.VMEM((8, 128), jnp.float32),
            pltpu.VMEM((N_PEERS, 8, 128), jnp.float32),
            pltpu.SemaphoreType.DMA((4,)),
            pltpu.SemaphoreType.DMA((4,)),
            pltpu.SemaphoreType.DMA((3,)),
            pltpu.SemaphoreType.DMA((4,)),
            pltpu.SemaphoreType.DMA((4,)),
            pltpu.SemaphoreType.DMA((4,)),
            pltpu.SemaphoreType.DMA((4,)),
            pltpu.SemaphoreType.DMA((4,)),
            pltpu.SemaphoreType.DMA((4,)),
            pltpu.SemaphoreType.DMA((N_PEERS,)),
            pltpu.SemaphoreType.DMA((N_PEERS,)),
            pltpu.SemaphoreType.REGULAR,
            pltpu.SemaphoreType.REGULAR,
        ],
        compiler_params=pltpu.CompilerParams(
            vmem_limit_bytes=63 * 1024 * 1024,
            collective_id=0,
        ),
    )(x, w_mat)
